# Initial kernel scaffold; baseline (speedup 1.0000x reference)
import jax
import jax.numpy as jnp
from jax.experimental import pallas as pl


def kernel(x, edge_index, W1, b1, W2, b2):
    raise NotImplementedError("write your pallas kernel here")


# speedup vs baseline: 5.6590x; 5.6590x over previous
import functools

import jax
import jax.numpy as jnp
from jax import lax
from jax.experimental import pallas as pl
from jax.experimental.pallas import tpu as pltpu
from jax.experimental.pallas import tpu_sc as plsc

N_NODES = 10000
N_EDGES = 320000
D_FEAT = 128
D_HID = 256
D_OUT = 256

NC = 2
NS = 16
NW = NC * NS
CHUNK = 128
CPT = -(-N_EDGES // (NW * CHUNK))
EPT = CPT * CHUNK
E_PAD = NW * EPT
ROWS_PAD = 10240
RPT = ROWS_PAD // NS
DEG_ROWS = ROWS_PAD // 128


def _sc_body(x_hbm, src_hbm, dst_hbm, zeros_hbm, zeros1_hbm,
             aggp_hbm, degp_hbm,
             src_v, dst_v, dstbuf, rows_v, ones_v, agg_sh, deg_sh,
             sem):
    c = lax.axis_index("c")
    s = lax.axis_index("s")
    wid = s * NC + c

    pltpu.sync_copy(zeros_hbm, agg_sh.at[pl.ds(s * RPT, RPT)])
    pltpu.sync_copy(zeros1_hbm.at[pl.ds(s * RPT, RPT)],
                    deg_sh.at[pl.ds(s * RPT, RPT)])

    base = wid * EPT
    pltpu.sync_copy(src_hbm.at[pl.ds(base, EPT)], src_v)
    pltpu.sync_copy(dst_hbm.at[pl.ds(base, EPT)], dst_v)

    ones = jnp.ones((16,), jnp.float32)
    for k in range(CHUNK // 16):
        ones_v[pl.ds(k * 16, 16)] = ones

    plsc.subcore_barrier()

    def step(i, carry):
        off = i * CHUNK
        for j in range(CHUNK // 16):
            dstbuf[pl.ds(j * 16, 16)] = dst_v[pl.ds(off + j * 16, 16)]
        pltpu.async_copy(x_hbm.at[src_v.at[pl.ds(off, CHUNK)]], rows_v,
                         sem).wait()
        pltpu.sync_copy(rows_v, agg_sh.at[dstbuf], add=True)
        pltpu.sync_copy(ones_v, deg_sh.at[dstbuf], add=True)
        return carry

    lax.fori_loop(0, CPT, step, 0)

    plsc.subcore_barrier()

    pltpu.sync_copy(agg_sh.at[pl.ds(s * RPT, RPT)],
                    aggp_hbm.at[c].at[pl.ds(s * RPT, RPT)])
    pltpu.sync_copy(deg_sh.at[pl.ds(s * RPT, RPT)],
                    degp_hbm.at[c].at[pl.ds(s * RPT, RPT)])


def _mlp_body(a0, a1, d0, d1, w1, b1, w2, b2, out):
    a = a0[0] + a1[0]
    d = d0[0] + d1[0]
    a = a / jnp.maximum(d, 1.0)
    h = jnp.dot(a, w1[...], preferred_element_type=jnp.float32) + b1[...]
    h = jnp.maximum(h, 0.0)
    out[...] = jnp.dot(h, w2[...], preferred_element_type=jnp.float32) + b2[...]


def kernel(x, edge_index, W1, b1, W2, b2):
    src = edge_index[0].astype(jnp.int32)
    dst = edge_index[1].astype(jnp.int32)
    pad = E_PAD - N_EDGES
    src = jnp.concatenate([src, jnp.zeros((pad,), jnp.int32)])
    dst = jnp.concatenate([dst, jnp.full((pad,), N_NODES, jnp.int32)])
    zeros = jnp.zeros((RPT, D_FEAT), jnp.float32)
    zeros1 = jnp.zeros((ROWS_PAD,), jnp.float32)

    mesh = plsc.VectorSubcoreMesh(core_axis_name="c", subcore_axis_name="s",
                                  num_cores=NC, num_subcores=NS)
    sc = pl.kernel(
        _sc_body,
        out_type=(
            jax.ShapeDtypeStruct((NC, ROWS_PAD, D_FEAT), jnp.float32),
            jax.ShapeDtypeStruct((NC, ROWS_PAD), jnp.float32),
        ),
        mesh=mesh,
        scratch_types=[
            pltpu.VMEM((EPT,), jnp.int32),
            pltpu.VMEM((EPT,), jnp.int32),
            pltpu.VMEM((CHUNK,), jnp.int32),
            pltpu.VMEM((CHUNK, D_FEAT), jnp.float32),
            pltpu.VMEM((CHUNK,), jnp.float32),
            pltpu.VMEM_SHARED((ROWS_PAD, D_FEAT), jnp.float32),
            pltpu.VMEM_SHARED((ROWS_PAD,), jnp.float32),
            pltpu.SemaphoreType.DMA,
        ],
    )
    aggp, degp = sc(x, src, dst, zeros, zeros1)
    degp = degp.reshape(NC, ROWS_PAD, 1)

    R = 400
    grid = (N_NODES // R,)
    out = pl.pallas_call(
        _mlp_body,
        grid=grid,
        in_specs=[
            pl.BlockSpec((1, R, D_FEAT), lambda i: (0, i, 0)),
            pl.BlockSpec((1, R, D_FEAT), lambda i: (1, i, 0)),
            pl.BlockSpec((1, R, 1), lambda i: (0, i, 0)),
            pl.BlockSpec((1, R, 1), lambda i: (1, i, 0)),
            pl.BlockSpec((D_FEAT, D_HID), lambda i: (0, 0)),
            pl.BlockSpec((1, D_HID), lambda i: (0, 0)),
            pl.BlockSpec((D_HID, D_OUT), lambda i: (0, 0)),
            pl.BlockSpec((1, D_OUT), lambda i: (0, 0)),
        ],
        out_specs=pl.BlockSpec((R, D_OUT), lambda i: (i, 0)),
        out_shape=jax.ShapeDtypeStruct((N_NODES, D_OUT), jnp.float32),
    )(aggp, aggp, degp, degp, W1, b1.reshape(1, D_HID), W2,
      b2.reshape(1, D_OUT))
    return out
